# Initial kernel scaffold; baseline (speedup 1.0000x reference)
import jax
import jax.numpy as jnp
from jax.experimental import pallas as pl


def kernel(x, edge_index, edge_attr, batch, Wn, bn0, We, be, Wf, bf, Ws, bs, gamma, beta, W1, b1, W2, b2, Wv, bv, Wen, ben, Wd, bd):
    raise NotImplementedError("write your pallas kernel here")


# speedup vs baseline: 1.2448x; 1.2448x over previous
import functools

import jax
import jax.numpy as jnp
from jax import lax
from jax.experimental import pallas as pl
from jax.experimental.pallas import tpu as pltpu
from jax.experimental.pallas import tpu_sc as plsc

N = 50000
E = 800000
G = 256
H = 96
L = 6

NB = 128
EB = 2048
N_PAD = 50176
NBLK = N_PAD // NB
E_PAD = 802816
NCHUNK = E_PAD // EB
SC_C = 128
BC = 128
PCH = 512
INV_STD = 1.0 / float(jnp.sqrt(jnp.float32(1.0 + 1e-5)))

_f32 = jnp.float32


def _bf(v):
    return v.astype(jnp.bfloat16).astype(_f32)


def _k_init(x_ref, w_ref, b_ref, o_ref):
    o_ref[...] = jnp.maximum(
        jnp.dot(_bf(x_ref[...]), w_ref[...], preferred_element_type=_f32)
        + b_ref[0:1, :], 0.0)


def _node_init(xp, wn8, bn0r):
    return pl.pallas_call(
        _k_init,
        grid=(N_PAD // 512,),
        in_specs=[pl.BlockSpec((512, 8), lambda i: (i, 0)),
                  pl.BlockSpec((8, BC), lambda i: (0, 0)),
                  pl.BlockSpec((8, BC), lambda i: (0, 0))],
        out_specs=pl.BlockSpec((512, BC), lambda i: (i, 0)),
        out_shape=jax.ShapeDtypeStruct((N_PAD, BC), _f32),
    )(xp, wn8, bn0r)


def _sc_gather_body(tbl_hbm, idx_hbm, out_hbm, idx_v, rows_v, sem):
    info = plsc.get_sparse_core_info()
    nc = info.num_cores
    wid = lax.axis_index("s") * nc + lax.axis_index("c")
    e_per_w = E_PAD // (nc * info.num_subcores)
    base = wid * e_per_w

    def body(k, carry):
        off = base + k * SC_C
        pltpu.sync_copy(idx_hbm.at[pl.ds(off, SC_C)], idx_v)
        pltpu.async_copy(tbl_hbm.at[idx_v], rows_v, sem).wait()
        pltpu.sync_copy(rows_v, out_hbm.at[pl.ds(off, SC_C)])
        return carry

    lax.fori_loop(0, e_per_w // SC_C, body, 0)


def _sc_gather(tbl, idx):
    mesh = plsc.VectorSubcoreMesh(core_axis_name="c", subcore_axis_name="s")
    fn = functools.partial(
        pl.kernel,
        out_type=jax.ShapeDtypeStruct((E_PAD, BC), _f32),
        mesh=mesh,
        scratch_types=[
            pltpu.VMEM((SC_C,), jnp.int32),
            pltpu.VMEM((SC_C, BC), _f32),
            pltpu.SemaphoreType.DMA,
        ],
    )(_sc_gather_body)
    return fn(tbl, idx)


def _k_main(bounds_ref, h_ref, wfd_ref, wsd_ref, wfe_ref, wse_ref,
            par_ref, part_ref, gb_hbm, meta_hbm, o_ref, gbuf, mbuf, agg,
            sem1, sem2):
    i = pl.program_id(0)
    lo = bounds_ref[i]
    hi = bounds_ref[i + 1]
    c0 = lo // EB
    c1 = (hi + EB - 1) // EB
    agg[...] = jnp.zeros((NB, H), _f32)
    blk_lo = i * NB

    def body(c, carry):
        cp1 = pltpu.make_async_copy(gb_hbm.at[pl.ds(c * EB, EB), :], gbuf,
                                    sem1)
        cp2 = pltpu.make_async_copy(meta_hbm.at[c], mbuf, sem2)
        cp1.start()
        cp2.start()
        cp1.wait()
        cp2.wait()
        dst_row = mbuf[0:1, :]
        a_row = mbuf[1:2, :]
        iot = lax.broadcasted_iota(jnp.int32, (NB, EB), 0) + blk_lo
        oht = (iot.astype(_f32) == dst_row).astype(_f32)
        pd = lax.dot_general(oht, h_ref[...], (((0,), (0,)), ((), ())),
                             preferred_element_type=_f32,
                             precision=lax.Precision.HIGHEST)
        z2 = jnp.concatenate([_bf(pd[:, :H]), _bf(gbuf[:, :H])], axis=1)
        xg = jnp.dot(z2, wfd_ref[...], preferred_element_type=_f32)
        xs = jnp.dot(z2, wsd_ref[...], preferred_element_type=_f32)
        e_t = jnp.maximum(part_ref[:, 0:1] * _bf(a_row) + part_ref[:, 1:2],
                          0.0)
        e_t = _bf(e_t)
        rf = lax.dot_general(e_t, wfe_ref[...], (((0,), (0,)), ((), ())),
                             preferred_element_type=_f32)
        rs = lax.dot_general(e_t, wse_ref[...], (((0,), (0,)), ((), ())),
                             preferred_element_type=_f32)
        xg = xg + rf + par_ref[2:3, :H]
        xs = xs + rs + par_ref[3:4, :H]
        gate = 1.0 / (1.0 + jnp.exp(-xg))
        core = jnp.maximum(xs, 0.0) + jnp.log(1.0 + jnp.exp(-jnp.abs(xs)))
        msg = gate * core
        agg[...] += jnp.dot(oht, msg, preferred_element_type=_f32,
                            precision=lax.Precision.HIGHEST)
        return carry

    lax.fori_loop(c0, c1, body, 0)
    aggp = jnp.concatenate([agg[...], jnp.zeros((NB, BC - H), _f32)], axis=1)
    hn = (h_ref[...] + aggp) * par_ref[0:1, :] + par_ref[1:2, :]
    o_ref[...] = jnp.maximum(hn, 0.0)


def _layer(bounds, h, gb, meta, wfd, wsd, wfe, wse, par, part):
    grid_spec = pltpu.PrefetchScalarGridSpec(
        num_scalar_prefetch=1,
        grid=(NBLK,),
        in_specs=[
            pl.BlockSpec((NB, BC), lambda i, *_: (i, 0)),
            pl.BlockSpec((2 * H, H), lambda i, *_: (0, 0)),
            pl.BlockSpec((2 * H, H), lambda i, *_: (0, 0)),
            pl.BlockSpec((H, H), lambda i, *_: (0, 0)),
            pl.BlockSpec((H, H), lambda i, *_: (0, 0)),
            pl.BlockSpec((8, BC), lambda i, *_: (0, 0)),
            pl.BlockSpec((H, 8), lambda i, *_: (0, 0)),
            pl.BlockSpec(memory_space=pl.ANY),
            pl.BlockSpec(memory_space=pl.ANY),
        ],
        out_specs=pl.BlockSpec((NB, BC), lambda i, *_: (i, 0)),
        scratch_shapes=[
            pltpu.VMEM((EB, BC), _f32),
            pltpu.VMEM((2, EB), _f32),
            pltpu.VMEM((NB, H), _f32),
            pltpu.SemaphoreType.DMA,
            pltpu.SemaphoreType.DMA,
        ],
    )
    return pl.pallas_call(
        _k_main,
        grid_spec=grid_spec,
        out_shape=jax.ShapeDtypeStruct((N_PAD, BC), _f32),
    )(bounds, h, wfd, wsd, wfe, wse, par, part, gb, meta)


def _k_pool(b2_ref, h_hbm, bat_hbm, sum_ref, max_ref, cnt_ref,
            hbuf, bbuf, sem1, sem2):
    g = pl.program_id(0)
    lo = b2_ref[g]
    hi = b2_ref[g + 1]
    c0 = lo // PCH
    c1 = (hi + PCH - 1) // PCH
    gf = g.astype(_f32)

    def body(c, carry):
        s_acc, m_acc, n_acc = carry
        cp1 = pltpu.make_async_copy(h_hbm.at[pl.ds(c * PCH, PCH), :], hbuf,
                                    sem1)
        cp2 = pltpu.make_async_copy(bat_hbm.at[c], bbuf, sem2)
        cp1.start()
        cp2.start()
        cp1.wait()
        cp2.wait()
        hv = hbuf[...]
        bcol = bbuf[:, 0:1]
        memb = bcol == gf
        mf = memb.astype(_f32)
        s_acc = s_acc + lax.dot_general(
            mf, hv, (((0,), (0,)), ((), ())), preferred_element_type=_f32,
            precision=lax.Precision.HIGHEST)
        m_acc = jnp.maximum(
            m_acc,
            jnp.max(jnp.where(memb, hv, -3.0e38), axis=0, keepdims=True))
        n_acc = n_acc + jnp.sum(mf)
        return (s_acc, m_acc, n_acc)

    s0 = jnp.zeros((1, BC), _f32)
    m0 = jnp.full((1, BC), -3.0e38, _f32)
    s_acc, m_acc, n_acc = lax.fori_loop(c0, c1, body, (s0, m0, _f32(0.0)))
    sum_ref[...] = s_acc.reshape(1, 1, BC)
    max_ref[...] = m_acc.reshape(1, 1, BC)
    cnt_ref[...] = jnp.broadcast_to(n_acc, (1, 1, BC))


def _pool(gbounds, h, batf):
    grid_spec = pltpu.PrefetchScalarGridSpec(
        num_scalar_prefetch=1,
        grid=(G,),
        in_specs=[
            pl.BlockSpec(memory_space=pl.ANY),
            pl.BlockSpec(memory_space=pl.ANY),
        ],
        out_specs=[
            pl.BlockSpec((1, 1, BC), lambda g, *_: (g, 0, 0)),
            pl.BlockSpec((1, 1, BC), lambda g, *_: (g, 0, 0)),
            pl.BlockSpec((1, 1, BC), lambda g, *_: (g, 0, 0)),
        ],
        scratch_shapes=[
            pltpu.VMEM((PCH, BC), _f32),
            pltpu.VMEM((PCH, 8), _f32),
            pltpu.SemaphoreType.DMA,
            pltpu.SemaphoreType.DMA,
        ],
    )
    return pl.pallas_call(
        _k_pool,
        grid_spec=grid_spec,
        out_shape=[jax.ShapeDtypeStruct((G, 1, BC), _f32),
                   jax.ShapeDtypeStruct((G, 1, BC), _f32),
                   jax.ShapeDtypeStruct((G, 1, BC), _f32)],
    )(gbounds, h, batf)


def _k_mlp(sum_ref, max_ref, cnt_ref, w1_ref, b1_ref, w2_ref, b2_ref,
           w3_ref, b3_ref, o_ref):
    ssum = sum_ref[...]
    cnt = cnt_ref[:, 0:1]
    mean = ssum / jnp.maximum(cnt, 1.0)
    smax = jnp.where(cnt > 0.5, max_ref[...], 0.0)
    g1 = jnp.concatenate([mean, smax, ssum], axis=1)
    g2 = jnp.maximum(jnp.dot(_bf(g1), w1_ref[...],
                             preferred_element_type=_f32)
                     + b1_ref[0:1, :], 0.0)
    g3 = jnp.maximum(jnp.dot(_bf(g2), w2_ref[...],
                             preferred_element_type=_f32)
                     + b2_ref[0:1, :], 0.0)
    o_ref[...] = (jnp.dot(_bf(g3), w3_ref[...],
                          preferred_element_type=_f32) + b3_ref[0:1, :])


def _mlp(ssum, smax, cnt, w1, b1r, w2, b2r, w3p, b3r):
    return pl.pallas_call(
        _k_mlp,
        grid=(1,),
        in_specs=[pl.BlockSpec((G, H), lambda i: (0, 0)),
                  pl.BlockSpec((G, H), lambda i: (0, 0)),
                  pl.BlockSpec((G, H), lambda i: (0, 0)),
                  pl.BlockSpec((3 * H, 2 * H), lambda i: (0, 0)),
                  pl.BlockSpec((8, 2 * H), lambda i: (0, 0)),
                  pl.BlockSpec((2 * H, H), lambda i: (0, 0)),
                  pl.BlockSpec((8, H), lambda i: (0, 0)),
                  pl.BlockSpec((H, 128), lambda i: (0, 0)),
                  pl.BlockSpec((8, 128), lambda i: (0, 0))],
        out_specs=pl.BlockSpec((G, 128), lambda i: (0, 0)),
        out_shape=jax.ShapeDtypeStruct((G, 128), _f32),
    )(ssum, smax, cnt, w1, b1r, w2, b2r, w3p, b3r)


def kernel(x, edge_index, edge_attr, batch, Wn, bn0, We, be, Wf, bf, Ws, bs,
           gamma, beta, W1, b1, W2, b2, Wv, bv, Wen, ben, Wd, bd):
    dst = edge_index[1]
    src = edge_index[0]
    a = edge_attr[:, 0]
    dst_s, src_s, a_s = lax.sort((dst, src, a), num_keys=1)
    pe = E_PAD - E
    dst_p = jnp.concatenate([dst_s, jnp.full((pe,), N_PAD, jnp.int32)])
    src_p = jnp.concatenate([src_s, jnp.zeros((pe,), jnp.int32)])
    a_p = jnp.concatenate([a_s, jnp.zeros((pe,), _f32)])
    bounds = jnp.searchsorted(
        dst_p, jnp.arange(NBLK + 1, dtype=jnp.int32) * NB).astype(jnp.int32)
    meta = jnp.stack([dst_p.astype(_f32).reshape(NCHUNK, EB),
                      a_p.reshape(NCHUNK, EB)], axis=1)

    xp = jnp.concatenate([x, jnp.zeros((N_PAD - N, 7), _f32)], axis=0)
    xp = jnp.concatenate([xp, jnp.ones((N_PAD, 1), _f32)], axis=1)
    bfr = lambda t: t.astype(jnp.bfloat16).astype(_f32)
    padc = lambda t: jnp.concatenate(
        [t, jnp.zeros((t.shape[0], BC - t.shape[1]), _f32)], axis=1)
    wn8 = padc(jnp.concatenate([bfr(Wn), jnp.zeros((1, H), _f32)], axis=0))
    bn0r = jnp.broadcast_to(padc(bn0[None, :]), (8, BC))

    batch_p = jnp.concatenate([batch, jnp.full((N_PAD - N,), G, jnp.int32)])
    gbounds = jnp.searchsorted(
        batch_p, jnp.arange(G + 1, dtype=jnp.int32)).astype(jnp.int32)
    batf = jnp.broadcast_to(
        batch_p.astype(_f32).reshape(N_PAD // PCH, PCH, 1),
        (N_PAD // PCH, PCH, 8))

    z96 = jnp.zeros((H,), _f32)
    wfds, wsds, wfes, wses, pars, parts = [], [], [], [], [], []
    for l in range(L):
        wfds.append(bfr(Wf[l][:2 * H]))
        wsds.append(bfr(Ws[l][:2 * H]))
        wfes.append(bfr(Wf[l][2 * H:]))
        wses.append(bfr(Ws[l][2 * H:]))
        pars.append(padc(jnp.stack(
            [gamma[l] * INV_STD, beta[l], bf[l], bs[l],
             z96, z96, z96, z96], axis=0)))
        parts.append(jnp.stack(
            [bfr(We[0]), be, z96, z96, z96, z96, z96, z96], axis=1))

    h = _node_init(xp, wn8, bn0r)
    for l in range(L):
        gbrows = _sc_gather(h, src_p)
        h = _layer(bounds, h, gbrows, meta,
                   wfds[l], wsds[l], wfes[l], wses[l], pars[l], parts[l])

    psum, pmax, pcnt = _pool(gbounds, h, batf)
    ssum = psum.reshape(G, BC)[:, :H]
    smax = pmax.reshape(G, BC)[:, :H]
    cnt = pcnt.reshape(G, BC)[:, :H]

    b1r = jnp.broadcast_to(b1[None, :], (8, 2 * H))
    b2r = jnp.broadcast_to(b2[None, :], (8, H))
    w3p = bfr(jnp.concatenate(
        [Wv, Wen, Wd, jnp.zeros((H, 125), _f32)], axis=1))
    b3r = jnp.broadcast_to(
        jnp.concatenate([bv, ben, bd, jnp.zeros((125,), _f32)])[None, :],
        (8, 128))
    out = _mlp(ssum, smax, cnt, bfr(W1), b1r, bfr(W2), b2r, w3p, b3r)
    return (out[:, 0:1], out[:, 1:2], out[:, 2:3])
